# baseline (device time: 36025 ns/iter reference)
import jax
import jax.numpy as jnp
from jax import lax
from jax.experimental import pallas as pl
from jax.experimental.pallas import tpu as pltpu

N_DEV = 4
SQ = 256
QROWS = SQ // N_DEV
D = 1024
HQ = 8
HKV = 2
DH = 128
GQ = HQ // HKV
GD = GQ * DH
SCALE = 0.08838834764831843


def kernel(x, Wq, Wo, K_ext, V_ext):
    skv = K_ext.shape[1]

    def body(x_ref, wq_ref, wo_ref, k_ref, v_ref, out_ref,
             loc_stage, part_comm, mlrow_stage, ml_comm,
             ag_stage, ag_comm,
             rs_send, rs_recv, ml_send, ml_recv, ag_send, ag_recv):
        my = lax.axis_index("i")
        left = (my + N_DEV - 1) % N_DEV
        right = (my + 1) % N_DEV
        diag = (my + 2) % N_DEV

        barrier = pltpu.get_barrier_semaphore()
        for nbr in (left, right):
            pl.semaphore_signal(barrier, inc=1, device_id=(nbr,),
                                device_id_type=pl.DeviceIdType.MESH)
        pl.semaphore_wait(barrier, 2)

        xb = x_ref[0].astype(jnp.bfloat16)
        wq = wq_ref[:].astype(jnp.bfloat16)
        q = lax.dot_general(xb, wq, (((1,), (0,)), ((), ())),
                            preferred_element_type=jnp.float32)
        q = q.astype(jnp.bfloat16)

        dests = (right, left, diag)

        o_loc, m_loc, l_loc, rdmas = {}, {}, {}, []
        for g in range(HKV):
            kg = k_ref[:, g * DH:(g + 1) * DH].astype(jnp.bfloat16)
            vg = v_ref[:, g * DH:(g + 1) * DH].astype(jnp.bfloat16)
            o_g, m_g, l_g = [], [], []
            for hh in range(GQ):
                h = g * GQ + hh
                qh = q[:, h * DH:(h + 1) * DH]
                s = lax.dot_general(qh, kg, (((1,), (1,)), ((), ())),
                                    preferred_element_type=jnp.float32) * SCALE
                mh = jnp.max(s, axis=1, keepdims=True)
                p = jnp.exp(s - mh)
                lh = jnp.sum(p, axis=1, keepdims=True)
                oh = lax.dot_general(p.astype(jnp.bfloat16), vg,
                                     (((1,), (0,)), ((), ())),
                                     preferred_element_type=jnp.float32)
                o_g.append(oh)
                m_g.append(mh)
                l_g.append(lh)

            m_loc[g] = jnp.concatenate(m_g, axis=1)
            l_loc[g] = jnp.concatenate(l_g, axis=1)
            o_loc[g] = o_g

            loc_stage[g] = jnp.concatenate(o_g, axis=1).astype(jnp.bfloat16)
            mlrow_stage[g] = jnp.concatenate(
                [m_loc[g], l_loc[g]], axis=1)

            for dest, dev in enumerate(dests):
                idx = dest * HKV + g
                o_r = pltpu.make_async_remote_copy(
                    src_ref=loc_stage.at[g, pl.ds(dev * QROWS, QROWS), :],
                    dst_ref=part_comm.at[dest, g],
                    send_sem=rs_send.at[idx], recv_sem=rs_recv.at[idx],
                    device_id=(dev,), device_id_type=pl.DeviceIdType.MESH)
                ml_r = pltpu.make_async_remote_copy(
                    src_ref=mlrow_stage.at[g, pl.ds(dev * QROWS, QROWS), :],
                    dst_ref=ml_comm.at[dest, g],
                    send_sem=ml_send.at[idx], recv_sem=ml_recv.at[idx],
                    device_id=(dev,), device_id_type=pl.DeviceIdType.MESH)
                o_r.start()
                ml_r.start()
                rdmas.append((o_r, ml_r))

        out = None
        wo = wo_ref[:].astype(jnp.bfloat16)
        for g in range(HKV):
            ml_q = mlrow_stage[g, pl.ds(my * QROWS, QROWS), :]
            m_acc = ml_q[:, 0:GQ]
            l_acc = ml_q[:, GQ:2 * GQ]
            o_q = loc_stage[g, pl.ds(my * QROWS, QROWS), :].astype(jnp.float32)
            o_acc = [o_q[:, hh * DH:(hh + 1) * DH] for hh in range(GQ)]
            for dest in range(3):
                o_r, ml_r = rdmas[g * 3 + dest]
                o_r.wait_recv()
                ml_r.wait_recv()
                ml_t = ml_comm[dest, g]
                m_r = ml_t[:, 0:GQ]
                l_r = ml_t[:, GQ:2 * GQ]
                m_new = jnp.maximum(m_acc, m_r)
                a_o = jnp.exp(m_acc - m_new)
                a_r = jnp.exp(m_r - m_new)
                l_acc = l_acc * a_o + l_r * a_r
                o_part = part_comm[dest, g].astype(jnp.float32)
                o_acc = [o_acc[hh] * a_o[:, hh:hh + 1]
                         + o_part[:, hh * DH:(hh + 1) * DH] * a_r[:, hh:hh + 1]
                         for hh in range(GQ)]
                m_acc = m_new
            attn_g = jnp.concatenate(
                [o_acc[hh] / l_acc[:, hh:hh + 1] for hh in range(GQ)], axis=1)
            part = lax.dot_general(attn_g.astype(jnp.bfloat16),
                                   wo[g * GD:(g + 1) * GD, :],
                                   (((1,), (0,)), ((), ())),
                                   preferred_element_type=jnp.float32)
            out = part if out is None else out + part

        out_ref[0, pl.ds(my * QROWS, QROWS), :] = out
        ag_stage[:] = out.astype(jnp.bfloat16)
        ag_rdmas = []
        for dest, dev in enumerate(dests):
            r = pltpu.make_async_remote_copy(
                src_ref=ag_stage, dst_ref=ag_comm.at[dest],
                send_sem=ag_send.at[dest], recv_sem=ag_recv.at[dest],
                device_id=(dev,), device_id_type=pl.DeviceIdType.MESH)
            r.start()
            ag_rdmas.append(r)

        for dest, origin in enumerate((left, right, diag)):
            ag_rdmas[dest].wait_recv()
            out_ref[0, pl.ds(origin * QROWS, QROWS), :] = (
                ag_comm[dest].astype(jnp.float32))

        for o_r, ml_r in rdmas:
            o_r.wait_send()
            ml_r.wait_send()
        for r in ag_rdmas:
            r.wait_send()

    K2 = K_ext.reshape(skv, HKV * DH)
    V2 = V_ext.reshape(skv, HKV * DH)

    return pl.pallas_call(
        body,
        out_shape=jax.ShapeDtypeStruct((1, SQ, D), jnp.float32),
        in_specs=[pl.BlockSpec(memory_space=pltpu.VMEM)] * 5,
        out_specs=pl.BlockSpec(memory_space=pltpu.VMEM),
        scratch_shapes=[
            pltpu.VMEM((HKV, SQ, GD), jnp.bfloat16),
            pltpu.VMEM((3, HKV, QROWS, GD), jnp.bfloat16),
            pltpu.VMEM((HKV, SQ, 2 * GQ), jnp.float32),
            pltpu.VMEM((3, HKV, QROWS, 2 * GQ), jnp.float32),
            pltpu.VMEM((QROWS, D), jnp.bfloat16),
            pltpu.VMEM((3, QROWS, D), jnp.bfloat16),
            pltpu.SemaphoreType.DMA((6,)),
            pltpu.SemaphoreType.DMA((6,)),
            pltpu.SemaphoreType.DMA((6,)),
            pltpu.SemaphoreType.DMA((6,)),
            pltpu.SemaphoreType.DMA((3,)),
            pltpu.SemaphoreType.DMA((3,)),
        ],
        compiler_params=pltpu.CompilerParams(collective_id=0),
    )(x, Wq, Wo, K2, V2)


# device time: 35681 ns/iter; 1.0096x vs baseline; 1.0096x over previous
import jax
import jax.numpy as jnp
from jax import lax
from jax.experimental import pallas as pl
from jax.experimental.pallas import tpu as pltpu

N_DEV = 4
SQ = 256
QROWS = SQ // N_DEV
D = 1024
HQ = 8
HKV = 2
DH = 128
GQ = HQ // HKV
GD = GQ * DH
SCALE = 0.08838834764831843


def kernel(x, Wq, Wo, K_ext, V_ext):
    skv = K_ext.shape[1]

    def body(x_ref, wq_ref, wo_ref, k_ref, v_ref, out_ref,
             loc_stage, part_comm, mlrow_stage,
             ag_stage, ag_comm,
             rs_send, rs_recv, ag_send, ag_recv):
        my = lax.axis_index("i")
        left = (my + N_DEV - 1) % N_DEV
        right = (my + 1) % N_DEV
        diag = (my + 2) % N_DEV

        barrier = pltpu.get_barrier_semaphore()
        for nbr in (left, right):
            pl.semaphore_signal(barrier, inc=1, device_id=(nbr,),
                                device_id_type=pl.DeviceIdType.MESH)
        pl.semaphore_wait(barrier, 2)

        xb = x_ref[0].astype(jnp.bfloat16)
        wq = wq_ref[:].astype(jnp.bfloat16)
        q = lax.dot_general(xb, wq, (((1,), (0,)), ((), ())),
                            preferred_element_type=jnp.float32)
        q = q.astype(jnp.bfloat16)

        dests = (right, left, diag)

        o_loc, m_loc, l_loc, rdmas = {}, {}, {}, []
        for g in range(HKV):
            kg = k_ref[:, g * DH:(g + 1) * DH].astype(jnp.bfloat16)
            vg = v_ref[:, g * DH:(g + 1) * DH].astype(jnp.bfloat16)
            o_g, m_g, l_g = [], [], []
            for hh in range(GQ):
                h = g * GQ + hh
                qh = q[:, h * DH:(h + 1) * DH]
                s = lax.dot_general(qh, kg, (((1,), (1,)), ((), ())),
                                    preferred_element_type=jnp.float32) * SCALE
                mh = jnp.max(s, axis=1, keepdims=True)
                p = jnp.exp(s - mh)
                lh = jnp.sum(p, axis=1, keepdims=True)
                oh = lax.dot_general(p.astype(jnp.bfloat16), vg,
                                     (((1,), (0,)), ((), ())),
                                     preferred_element_type=jnp.float32)
                o_g.append(oh)
                m_g.append(mh)
                l_g.append(lh)

            m_loc[g] = jnp.concatenate(m_g, axis=1)
            l_loc[g] = jnp.concatenate(l_g, axis=1)
            o_loc[g] = o_g

            ml_row = jnp.concatenate([m_loc[g], l_loc[g]], axis=1)
            mlrow_stage[g] = ml_row
            loc_stage[g] = jnp.concatenate(
                o_g + [ml_row], axis=1).astype(jnp.bfloat16)

            for dest, dev in enumerate(dests):
                idx = dest * HKV + g
                o_r = pltpu.make_async_remote_copy(
                    src_ref=loc_stage.at[g, pl.ds(dev * QROWS, QROWS), :],
                    dst_ref=part_comm.at[dest, g],
                    send_sem=rs_send.at[idx], recv_sem=rs_recv.at[idx],
                    device_id=(dev,), device_id_type=pl.DeviceIdType.MESH)
                o_r.start()
                rdmas.append(o_r)

        out = None
        wo = wo_ref[:].astype(jnp.bfloat16)
        for g in range(HKV):
            ml_q = mlrow_stage[g, pl.ds(my * QROWS, QROWS), :]
            m_acc = ml_q[:, 0:GQ]
            l_acc = ml_q[:, GQ:2 * GQ]
            o_q = loc_stage[g, pl.ds(my * QROWS, QROWS), :].astype(jnp.float32)
            o_acc = [o_q[:, hh * DH:(hh + 1) * DH] for hh in range(GQ)]
            for dest in range(3):
                o_r = rdmas[g * 3 + dest]
                o_r.wait_recv()
                blk = part_comm[dest, g].astype(jnp.float32)
                ml_t = blk[:, HQ * DH // HKV:]
                m_r = ml_t[:, 0:GQ]
                l_r = ml_t[:, GQ:2 * GQ]
                m_new = jnp.maximum(m_acc, m_r)
                a_o = jnp.exp(m_acc - m_new)
                a_r = jnp.exp(m_r - m_new)
                l_acc = l_acc * a_o + l_r * a_r
                o_part = blk
                o_acc = [o_acc[hh] * a_o[:, hh:hh + 1]
                         + o_part[:, hh * DH:(hh + 1) * DH] * a_r[:, hh:hh + 1]
                         for hh in range(GQ)]
                m_acc = m_new
            attn_g = jnp.concatenate(
                [o_acc[hh] / l_acc[:, hh:hh + 1] for hh in range(GQ)], axis=1)
            part = lax.dot_general(attn_g.astype(jnp.bfloat16),
                                   wo[g * GD:(g + 1) * GD, :],
                                   (((1,), (0,)), ((), ())),
                                   preferred_element_type=jnp.float32)
            out = part if out is None else out + part

        out_ref[0, pl.ds(my * QROWS, QROWS), :] = out
        ag_stage[:] = out.astype(jnp.bfloat16)
        ag_rdmas = []
        for dest, dev in enumerate(dests):
            r = pltpu.make_async_remote_copy(
                src_ref=ag_stage, dst_ref=ag_comm.at[dest],
                send_sem=ag_send.at[dest], recv_sem=ag_recv.at[dest],
                device_id=(dev,), device_id_type=pl.DeviceIdType.MESH)
            r.start()
            ag_rdmas.append(r)

        for dest, origin in enumerate((left, right, diag)):
            ag_rdmas[dest].wait_recv()
            out_ref[0, pl.ds(origin * QROWS, QROWS), :] = (
                ag_comm[dest].astype(jnp.float32))

        for o_r in rdmas:
            o_r.wait_send()
        for r in ag_rdmas:
            r.wait_send()

    K2 = K_ext.reshape(skv, HKV * DH)
    V2 = V_ext.reshape(skv, HKV * DH)

    return pl.pallas_call(
        body,
        out_shape=jax.ShapeDtypeStruct((1, SQ, D), jnp.float32),
        in_specs=[pl.BlockSpec(memory_space=pltpu.VMEM)] * 5,
        out_specs=pl.BlockSpec(memory_space=pltpu.VMEM),
        scratch_shapes=[
            pltpu.VMEM((HKV, SQ, GD + 2 * GQ), jnp.bfloat16),
            pltpu.VMEM((3, HKV, QROWS, GD + 2 * GQ), jnp.bfloat16),
            pltpu.VMEM((HKV, SQ, 2 * GQ), jnp.float32),
            pltpu.VMEM((QROWS, D), jnp.bfloat16),
            pltpu.VMEM((3, QROWS, D), jnp.bfloat16),
            pltpu.SemaphoreType.DMA((6,)),
            pltpu.SemaphoreType.DMA((6,)),
            pltpu.SemaphoreType.DMA((3,)),
            pltpu.SemaphoreType.DMA((3,)),
        ],
        compiler_params=pltpu.CompilerParams(collective_id=0),
    )(x, Wq, Wo, K2, V2)
